# baseline (speedup 1.0000x reference)
import jax
import jax.numpy as jnp
from jax import lax
from jax.experimental import pallas as pl
from jax.experimental.pallas import tpu as pltpu
from jax.experimental.pallas import tpu_sc as plsc

B, D, H, W = 8, 256, 24, 24
N = B * H * W
V = 8192
KT = 512
NK = V // KT
TT = 512
NT = N // TT

_BIG = float(2**30)


_RB = 8


def _argmin_body(ht2_ref, cb_ref, fn_ref, cn_ref, out_ref):
    t = pl.program_id(0)
    ht2 = ht2_ref[...]
    fn = fn_ref[pl.ds(t, 1), :]
    acc_v = jnp.full((_RB, TT), jnp.inf, dtype=jnp.float32)
    acc_b = jnp.zeros((_RB, TT), dtype=jnp.float32)
    for k in range(NK):
        cbk = cb_ref[pl.ds(k * KT, KT), :]
        mm2 = lax.dot_general(cbk, ht2, (((1,), (0,)), ((), ())),
                              preferred_element_type=jnp.float32)
        cn = cn_ref[pl.ds(k * KT, KT), :]
        for r in range(KT // _RB):
            d2 = (fn + mm2[r * _RB:(r + 1) * _RB, :]) + cn[r * _RB:(r + 1) * _RB, :]
            upd = d2 < acc_v
            acc_v = jnp.where(upd, d2, acc_v)
            acc_b = jnp.where(upd, float(k * (KT // _RB) + r), acc_b)
    sub_iota = lax.broadcasted_iota(jnp.int32, (_RB, TT), 0).astype(jnp.float32)
    v, i = acc_v, acc_b * float(_RB) + sub_iota
    for s in (4, 2, 1):
        v1, v2 = v[:s], v[s:]
        i1, i2 = i[:s], i[s:]
        take2 = (v2 < v1) | ((v2 == v1) & (i2 < i1))
        v = jnp.where(take2, v2, v1)
        i = jnp.where(take2, i2, i1)
    out_ref[pl.ds(t, 1), :] = i.astype(jnp.int32)


def _nearest_codes(ht, cb, fn, cn):
    return pl.pallas_call(
        _argmin_body,
        grid=(NT,),
        in_specs=[
            pl.BlockSpec((D, TT), lambda t: (0, t)),
            pl.BlockSpec((V, D), lambda t: (0, 0)),
            pl.BlockSpec((NT, TT), lambda t: (0, 0)),
            pl.BlockSpec((V, 1), lambda t: (0, 0)),
        ],
        out_specs=pl.BlockSpec((NT, TT), lambda t: (0, 0)),
        out_shape=jax.ShapeDtypeStruct((NT, TT), jnp.int32),
        compiler_params=pltpu.CompilerParams(
            dimension_semantics=("arbitrary",)),
    )(ht, cb, fn, cn)


_NC = 2
_NS = 16
_NW = _NC * _NS
_ROWS_PER_W = N // _NW
_CHUNK = 72
_NCHUNK = _ROWS_PER_W // _CHUNK


def _gather_body(idx_hbm, cb_hbm, out_hbm, idx_v, rows_v, sem):
    wid = lax.axis_index("s") * _NC + lax.axis_index("c")
    base = wid * _ROWS_PER_W
    for c in range(_NCHUNK):
        pltpu.sync_copy(idx_hbm.at[pl.ds(base + c * _CHUNK, _CHUNK)],
                        idx_v.at[c])
        pltpu.async_copy(cb_hbm.at[idx_v.at[c]], rows_v, sem).wait()
        pltpu.sync_copy(rows_v, out_hbm.at[pl.ds(base + c * _CHUNK, _CHUNK)])


def _gather_rows(idx_flat, cb):
    return pl.kernel(
        _gather_body,
        mesh=plsc.VectorSubcoreMesh(core_axis_name="c", subcore_axis_name="s"),
        out_type=jax.ShapeDtypeStruct((N, D), jnp.float32),
        scratch_types=[
            pltpu.VMEM((_NCHUNK, _CHUNK), jnp.int32),
            pltpu.VMEM((_CHUNK, D), jnp.float32),
            pltpu.SemaphoreType.DMA,
        ],
    )(idx_flat, cb)


def kernel(h, cb):
    flat = jnp.transpose(h, (0, 2, 3, 1)).reshape(N, D)
    ht2 = -2.0 * jnp.transpose(h.reshape(B, D, H * W), (1, 0, 2)).reshape(D, N)
    fn = jnp.sum(flat * flat, axis=1).reshape(NT, TT)
    cn = jnp.sum(cb * cb, axis=1).reshape(V, 1)
    idx = _nearest_codes(ht2, cb, fn, cn)
    idx_flat = idx.reshape(N)
    q = _gather_rows(idx_flat, cb)
    z = idx_flat.reshape(B, H, W)
    return (z, q.reshape(B, H, W, D))


# speedup vs baseline: 1.0159x; 1.0159x over previous
import jax
import jax.numpy as jnp
from jax import lax
from jax.experimental import pallas as pl
from jax.experimental.pallas import tpu as pltpu
from jax.experimental.pallas import tpu_sc as plsc

B, D, H, W = 8, 256, 24, 24
N = B * H * W
V = 8192
KT = 512
NK = V // KT
TT = 512
NT = N // TT

_BIG = float(2**30)


_RB = 8


def _argmin_body(ft2_ref, cb_ref, fn_ref, cn_ref, out_ref):
    t = pl.program_id(0)
    ft2 = ft2_ref[...]
    fn = fn_ref[pl.ds(t, 1), :]
    acc_v = jnp.full((_RB, TT), jnp.inf, dtype=jnp.float32)
    acc_b = jnp.zeros((_RB, TT), dtype=jnp.float32)
    for k in range(NK):
        cbk = cb_ref[pl.ds(k * KT, KT), :]
        mm2 = lax.dot_general(cbk, ft2, (((1,), (1,)), ((), ())),
                              preferred_element_type=jnp.float32)
        cn = cn_ref[pl.ds(k * KT, KT), :]
        for r in range(KT // _RB):
            d2 = (fn + mm2[r * _RB:(r + 1) * _RB, :]) + cn[r * _RB:(r + 1) * _RB, :]
            upd = d2 < acc_v
            acc_v = jnp.where(upd, d2, acc_v)
            acc_b = jnp.where(upd, float(k * (KT // _RB) + r), acc_b)
    sub_iota = lax.broadcasted_iota(jnp.int32, (_RB, TT), 0).astype(jnp.float32)
    v, i = acc_v, acc_b * float(_RB) + sub_iota
    for s in (4, 2, 1):
        v1, v2 = v[:s], v[s:]
        i1, i2 = i[:s], i[s:]
        take2 = (v2 < v1) | ((v2 == v1) & (i2 < i1))
        v = jnp.where(take2, v2, v1)
        i = jnp.where(take2, i2, i1)
    out_ref[pl.ds(t, 1), :] = i.astype(jnp.int32)


def _nearest_codes(ht, cb, fn, cn):
    return pl.pallas_call(
        _argmin_body,
        grid=(NT,),
        in_specs=[
            pl.BlockSpec((TT, D), lambda t: (t, 0)),
            pl.BlockSpec((V, D), lambda t: (0, 0)),
            pl.BlockSpec((NT, TT), lambda t: (0, 0)),
            pl.BlockSpec((V, 1), lambda t: (0, 0)),
        ],
        out_specs=pl.BlockSpec((NT, TT), lambda t: (0, 0)),
        out_shape=jax.ShapeDtypeStruct((NT, TT), jnp.int32),
        compiler_params=pltpu.CompilerParams(
            dimension_semantics=("arbitrary",)),
    )(ht, cb, fn, cn)


_NC = 2
_NS = 16
_NW = _NC * _NS
_ROWS_PER_W = N // _NW
_CHUNK = 72
_NCHUNK = _ROWS_PER_W // _CHUNK


def _gather_body(idx_hbm, cb_hbm, out_hbm, idx_v, rows_v, sem):
    wid = lax.axis_index("s") * _NC + lax.axis_index("c")
    base = wid * _ROWS_PER_W
    for c in range(_NCHUNK):
        pltpu.sync_copy(idx_hbm.at[pl.ds(base + c * _CHUNK, _CHUNK)],
                        idx_v.at[c])
        pltpu.async_copy(cb_hbm.at[idx_v.at[c]], rows_v, sem).wait()
        pltpu.sync_copy(rows_v, out_hbm.at[pl.ds(base + c * _CHUNK, _CHUNK)])


def _gather_rows(idx_flat, cb):
    return pl.kernel(
        _gather_body,
        mesh=plsc.VectorSubcoreMesh(core_axis_name="c", subcore_axis_name="s"),
        out_type=jax.ShapeDtypeStruct((N, D), jnp.float32),
        scratch_types=[
            pltpu.VMEM((_NCHUNK, _CHUNK), jnp.int32),
            pltpu.VMEM((_CHUNK, D), jnp.float32),
            pltpu.SemaphoreType.DMA,
        ],
    )(idx_flat, cb)


def kernel(h, cb):
    flat = jnp.transpose(h, (0, 2, 3, 1)).reshape(N, D)
    ft2 = -2.0 * flat
    fn = jnp.sum(flat * flat, axis=1).reshape(NT, TT)
    cn = jnp.sum(cb * cb, axis=1).reshape(V, 1)
    idx = _nearest_codes(ft2, cb, fn, cn)
    idx_flat = idx.reshape(N)
    q = _gather_rows(idx_flat, cb)
    z = idx_flat.reshape(B, H, W)
    return (z, q.reshape(B, H, W, D))


# speedup vs baseline: 1.0195x; 1.0036x over previous
import jax
import jax.numpy as jnp
from jax import lax
from jax.experimental import pallas as pl
from jax.experimental.pallas import tpu as pltpu
from jax.experimental.pallas import tpu_sc as plsc

B, D, H, W = 8, 256, 24, 24
N = B * H * W
V = 8192
KT = 512
NK = V // KT
TT = 512
NT = N // TT

_BIG = float(2**30)


_RB = 8


def _argmin_body(ft2_ref, cb_ref, fn_ref, cn_ref, out_ref):
    t = pl.program_id(0)
    ft2 = ft2_ref[...]
    fn = fn_ref[pl.ds(t, 1), :]
    acc_v = jnp.full((_RB, TT), jnp.inf, dtype=jnp.float32)
    acc_b = jnp.zeros((_RB, TT), dtype=jnp.float32)
    for k in range(NK):
        cbk = cb_ref[pl.ds(k * KT, KT), :]
        mm2 = lax.dot_general(cbk, ft2, (((1,), (1,)), ((), ())),
                              preferred_element_type=jnp.float32)
        cn = cn_ref[pl.ds(k * KT, KT), :]
        for r in range(KT // _RB):
            d2 = (fn + mm2[r * _RB:(r + 1) * _RB, :]) + cn[r * _RB:(r + 1) * _RB, :]
            upd = d2 < acc_v
            acc_v = jnp.where(upd, d2, acc_v)
            acc_b = jnp.where(upd, float(k * (KT // _RB) + r), acc_b)
    sub_iota = lax.broadcasted_iota(jnp.int32, (_RB, TT), 0).astype(jnp.float32)
    v, i = acc_v, acc_b * float(_RB) + sub_iota
    for s in (4, 2, 1):
        v1, v2 = v[:s], v[s:]
        i1, i2 = i[:s], i[s:]
        take2 = (v2 < v1) | ((v2 == v1) & (i2 < i1))
        v = jnp.where(take2, v2, v1)
        i = jnp.where(take2, i2, i1)
    out_ref[pl.ds(t, 1), :] = i.astype(jnp.int32)


def _nearest_codes(ht, cb, fn, cn):
    return pl.pallas_call(
        _argmin_body,
        grid=(NT,),
        in_specs=[
            pl.BlockSpec((TT, D), lambda t: (t, 0)),
            pl.BlockSpec((V, D), lambda t: (0, 0)),
            pl.BlockSpec((NT, TT), lambda t: (0, 0)),
            pl.BlockSpec((V, 1), lambda t: (0, 0)),
        ],
        out_specs=pl.BlockSpec((NT, TT), lambda t: (0, 0)),
        out_shape=jax.ShapeDtypeStruct((NT, TT), jnp.int32),
        compiler_params=pltpu.CompilerParams(
            dimension_semantics=("arbitrary",)),
    )(ht, cb, fn, cn)


_NC = 2
_NS = 16
_NW = _NC * _NS
_ROWS_PER_W = N // _NW
_CHUNK = 72
_NCHUNK = _ROWS_PER_W // _CHUNK


def _gather_body(idx_hbm, cb_hbm, out_hbm, idx_v, rows_v, sem):
    wid = lax.axis_index("s") * _NC + lax.axis_index("c")
    base = wid * _ROWS_PER_W
    for c in range(_NCHUNK):
        pltpu.sync_copy(idx_hbm.at[pl.ds(base + c * _CHUNK, _CHUNK)],
                        idx_v.at[c])
        pltpu.async_copy(cb_hbm.at[idx_v.at[c]], rows_v, sem).wait()
        pltpu.sync_copy(rows_v, out_hbm.at[pl.ds(base + c * _CHUNK, _CHUNK)])


def _gather_rows(idx_flat, cb):
    return pl.kernel(
        _gather_body,
        mesh=plsc.VectorSubcoreMesh(core_axis_name="c", subcore_axis_name="s"),
        out_type=jax.ShapeDtypeStruct((N, D), jnp.float32),
        scratch_types=[
            pltpu.VMEM((_NCHUNK, _CHUNK), jnp.int32),
            pltpu.VMEM((_CHUNK, D), jnp.float32),
            pltpu.SemaphoreType.DMA,
        ],
    )(idx_flat, cb)


def kernel(h, cb):
    ft2 = -2.0 * jnp.transpose(h, (0, 2, 3, 1)).reshape(N, D)
    fn = (jnp.sum(ft2 * ft2, axis=1) * 0.25).reshape(NT, TT)
    cn = jnp.sum(cb * cb, axis=1).reshape(V, 1)
    idx = _nearest_codes(ft2, cb, fn, cn)
    idx_flat = idx.reshape(N)
    q = _gather_rows(idx_flat, cb)
    z = idx_flat.reshape(B, H, W)
    return (z, q.reshape(B, H, W, D))


# speedup vs baseline: 1.0267x; 1.0070x over previous
import jax
import jax.numpy as jnp
from jax import lax
from jax.experimental import pallas as pl
from jax.experimental.pallas import tpu as pltpu
from jax.experimental.pallas import tpu_sc as plsc

B, D, H, W = 8, 256, 24, 24
N = B * H * W
V = 8192
KT = 512
NK = V // KT
TT = 512
NT = N // TT

_BIG = float(2**30)


_RB = 8


def _argmin_body(ft2_ref, cb_ref, fn_ref, cn_ref, out_ref):
    t = pl.program_id(0)
    ft2 = ft2_ref[...]
    fn = fn_ref[pl.ds(t, 1), :]
    acc_v = jnp.full((_RB, TT), jnp.inf, dtype=jnp.float32)
    acc_b = jnp.zeros((_RB, TT), dtype=jnp.float32)
    for k in range(NK):
        cbk = cb_ref[pl.ds(k * KT, KT), :]
        mm2 = lax.dot_general(cbk, ft2, (((1,), (1,)), ((), ())),
                              preferred_element_type=jnp.float32)
        cn = cn_ref[pl.ds(k * KT, KT), :]
        for r in range(KT // _RB):
            d2 = (fn + mm2[r * _RB:(r + 1) * _RB, :]) + cn[r * _RB:(r + 1) * _RB, :]
            upd = d2 < acc_v
            acc_v = jnp.where(upd, d2, acc_v)
            acc_b = jnp.where(upd, float(k * (KT // _RB) + r), acc_b)
    sub_iota = lax.broadcasted_iota(jnp.int32, (_RB, TT), 0).astype(jnp.float32)
    v, i = acc_v, acc_b * float(_RB) + sub_iota
    for s in (4, 2, 1):
        v1, v2 = v[:s], v[s:]
        i1, i2 = i[:s], i[s:]
        take2 = (v2 < v1) | ((v2 == v1) & (i2 < i1))
        v = jnp.where(take2, v2, v1)
        i = jnp.where(take2, i2, i1)
    out_ref[pl.ds(t, 1), :] = i.astype(jnp.int32)


def _nearest_codes(ht, cb, fn, cn):
    return pl.pallas_call(
        _argmin_body,
        grid=(NT,),
        in_specs=[
            pl.BlockSpec((TT, D), lambda t: (t, 0)),
            pl.BlockSpec((V, D), lambda t: (0, 0)),
            pl.BlockSpec((NT, TT), lambda t: (0, 0)),
            pl.BlockSpec((V, 1), lambda t: (0, 0)),
        ],
        out_specs=pl.BlockSpec((NT, TT), lambda t: (0, 0)),
        out_shape=jax.ShapeDtypeStruct((NT, TT), jnp.int32),
        compiler_params=pltpu.CompilerParams(
            dimension_semantics=("arbitrary",)),
    )(ht, cb, fn, cn)


_NC = 2
_NS = 16
_NW = _NC * _NS
_ROWS_PER_W = N // _NW
_CHUNK = 72
_NCHUNK = _ROWS_PER_W // _CHUNK


def _gather_body(idx_hbm, cb_hbm, out_hbm, idx_v, rows_v, sem0, sem1):
    wid = lax.axis_index("s") * _NC + lax.axis_index("c")
    base = wid * _ROWS_PER_W
    sems = (sem0, sem1)
    for c in range(_NCHUNK):
        pltpu.sync_copy(idx_hbm.at[pl.ds(base + c * _CHUNK, _CHUNK)],
                        idx_v.at[c])
    prev = pltpu.async_copy(cb_hbm.at[idx_v.at[0]], rows_v.at[0], sems[0])
    for c in range(_NCHUNK):
        nxt = None
        if c + 1 < _NCHUNK:
            nxt = pltpu.async_copy(cb_hbm.at[idx_v.at[c + 1]],
                                   rows_v.at[(c + 1) % 2], sems[(c + 1) % 2])
        prev.wait()
        pltpu.sync_copy(rows_v.at[c % 2],
                        out_hbm.at[pl.ds(base + c * _CHUNK, _CHUNK)])
        prev = nxt


def _gather_rows(idx_flat, cb):
    return pl.kernel(
        _gather_body,
        mesh=plsc.VectorSubcoreMesh(core_axis_name="c", subcore_axis_name="s"),
        out_type=jax.ShapeDtypeStruct((N, D), jnp.float32),
        scratch_types=[
            pltpu.VMEM((_NCHUNK, _CHUNK), jnp.int32),
            pltpu.VMEM((2, _CHUNK, D), jnp.float32),
            pltpu.SemaphoreType.DMA,
            pltpu.SemaphoreType.DMA,
        ],
    )(idx_flat, cb)


def kernel(h, cb):
    ft2 = -2.0 * jnp.transpose(h, (0, 2, 3, 1)).reshape(N, D)
    fn = (jnp.sum(ft2 * ft2, axis=1) * 0.25).reshape(NT, TT)
    cn = jnp.sum(cb * cb, axis=1).reshape(V, 1)
    idx = _nearest_codes(ft2, cb, fn, cn)
    idx_flat = idx.reshape(N)
    q = _gather_rows(idx_flat, cb)
    z = idx_flat.reshape(B, H, W)
    return (z, q.reshape(B, H, W, D))


# speedup vs baseline: 1.0311x; 1.0043x over previous
import jax
import jax.numpy as jnp
from jax import lax
from jax.experimental import pallas as pl
from jax.experimental.pallas import tpu as pltpu
from jax.experimental.pallas import tpu_sc as plsc

B, D, H, W = 8, 256, 24, 24
N = B * H * W
V = 8192
KT = 512
NK = V // KT
TT = 768
NT = N // TT

_BIG = float(2**30)


_RB = 8


def _argmin_body(ft2_ref, cb_ref, fn_ref, cn_ref, out_ref):
    t = pl.program_id(0)
    ft2 = ft2_ref[...]
    fn = fn_ref[pl.ds(t, 1), :]
    acc_v = jnp.full((_RB, TT), jnp.inf, dtype=jnp.float32)
    acc_b = jnp.zeros((_RB, TT), dtype=jnp.float32)
    for k in range(NK):
        cbk = cb_ref[pl.ds(k * KT, KT), :]
        mm2 = lax.dot_general(cbk, ft2, (((1,), (1,)), ((), ())),
                              preferred_element_type=jnp.float32)
        cn = cn_ref[pl.ds(k * KT, KT), :]
        for r in range(KT // _RB):
            d2 = (fn + mm2[r * _RB:(r + 1) * _RB, :]) + cn[r * _RB:(r + 1) * _RB, :]
            upd = d2 < acc_v
            acc_v = jnp.where(upd, d2, acc_v)
            acc_b = jnp.where(upd, float(k * (KT // _RB) + r), acc_b)
    sub_iota = lax.broadcasted_iota(jnp.int32, (_RB, TT), 0).astype(jnp.float32)
    v, i = acc_v, acc_b * float(_RB) + sub_iota
    for s in (4, 2, 1):
        v1, v2 = v[:s], v[s:]
        i1, i2 = i[:s], i[s:]
        take2 = (v2 < v1) | ((v2 == v1) & (i2 < i1))
        v = jnp.where(take2, v2, v1)
        i = jnp.where(take2, i2, i1)
    out_ref[pl.ds(t, 1), :] = i.astype(jnp.int32)


def _nearest_codes(ht, cb, fn, cn):
    return pl.pallas_call(
        _argmin_body,
        grid=(NT,),
        in_specs=[
            pl.BlockSpec((TT, D), lambda t: (t, 0)),
            pl.BlockSpec((V, D), lambda t: (0, 0)),
            pl.BlockSpec((NT, TT), lambda t: (0, 0)),
            pl.BlockSpec((V, 1), lambda t: (0, 0)),
        ],
        out_specs=pl.BlockSpec((NT, TT), lambda t: (0, 0)),
        out_shape=jax.ShapeDtypeStruct((NT, TT), jnp.int32),
        compiler_params=pltpu.CompilerParams(
            dimension_semantics=("arbitrary",)),
    )(ht, cb, fn, cn)


_NC = 2
_NS = 16
_NW = _NC * _NS
_ROWS_PER_W = N // _NW
_CHUNK = 72
_NCHUNK = _ROWS_PER_W // _CHUNK


def _gather_body(idx_hbm, cb_hbm, out_hbm, idx_v, rows_v, sem0, sem1):
    wid = lax.axis_index("s") * _NC + lax.axis_index("c")
    base = wid * _ROWS_PER_W
    sems = (sem0, sem1)
    for c in range(_NCHUNK):
        pltpu.sync_copy(idx_hbm.at[pl.ds(base + c * _CHUNK, _CHUNK)],
                        idx_v.at[c])
    prev = pltpu.async_copy(cb_hbm.at[idx_v.at[0]], rows_v.at[0], sems[0])
    for c in range(_NCHUNK):
        nxt = None
        if c + 1 < _NCHUNK:
            nxt = pltpu.async_copy(cb_hbm.at[idx_v.at[c + 1]],
                                   rows_v.at[(c + 1) % 2], sems[(c + 1) % 2])
        prev.wait()
        pltpu.sync_copy(rows_v.at[c % 2],
                        out_hbm.at[pl.ds(base + c * _CHUNK, _CHUNK)])
        prev = nxt


def _gather_rows(idx_flat, cb):
    return pl.kernel(
        _gather_body,
        mesh=plsc.VectorSubcoreMesh(core_axis_name="c", subcore_axis_name="s"),
        out_type=jax.ShapeDtypeStruct((N, D), jnp.float32),
        scratch_types=[
            pltpu.VMEM((_NCHUNK, _CHUNK), jnp.int32),
            pltpu.VMEM((2, _CHUNK, D), jnp.float32),
            pltpu.SemaphoreType.DMA,
            pltpu.SemaphoreType.DMA,
        ],
    )(idx_flat, cb)


def kernel(h, cb):
    ft2 = -2.0 * jnp.transpose(h, (0, 2, 3, 1)).reshape(N, D)
    fn = (jnp.sum(ft2 * ft2, axis=1) * 0.25).reshape(NT, TT)
    cn = jnp.sum(cb * cb, axis=1).reshape(V, 1)
    idx = _nearest_codes(ft2, cb, fn, cn)
    idx_flat = idx.reshape(N)
    q = _gather_rows(idx_flat, cb)
    z = idx_flat.reshape(B, H, W)
    return (z, q.reshape(B, H, W, D))


# speedup vs baseline: 1.1439x; 1.1094x over previous
import jax
import jax.numpy as jnp
from jax import lax
from jax.experimental import pallas as pl
from jax.experimental.pallas import tpu as pltpu
from jax.experimental.pallas import tpu_sc as plsc

B, D, H, W = 8, 256, 24, 24
N = B * H * W
V = 8192
KT = 512
NK = V // KT
TT = 768
NT = N // TT

_BIG = float(2**30)


_RB = 8


def _argmin_body(ft2_ref, cb_ref, fn_ref, out_ref, cn_ref):
    t = pl.program_id(0)

    @pl.when(t == 0)
    def _cn():
        for k in range(NK):
            cbk = cb_ref[pl.ds(k * KT, KT), :]
            cn_ref[pl.ds(k * KT, KT), :] = jnp.sum(
                cbk * cbk, axis=1, keepdims=True)

    ft2 = ft2_ref[...]
    fn = fn_ref[pl.ds(t, 1), :]
    acc_v = jnp.full((_RB, TT), jnp.inf, dtype=jnp.float32)
    acc_b = jnp.zeros((_RB, TT), dtype=jnp.float32)
    for k in range(NK):
        cbk = cb_ref[pl.ds(k * KT, KT), :]
        mm2 = lax.dot_general(cbk, ft2, (((1,), (1,)), ((), ())),
                              preferred_element_type=jnp.float32)
        cn = cn_ref[pl.ds(k * KT, KT), :]
        for r in range(KT // _RB):
            d2 = (fn + mm2[r * _RB:(r + 1) * _RB, :]) + cn[r * _RB:(r + 1) * _RB, :]
            upd = d2 < acc_v
            acc_v = jnp.where(upd, d2, acc_v)
            acc_b = jnp.where(upd, float(k * (KT // _RB) + r), acc_b)
    sub_iota = lax.broadcasted_iota(jnp.int32, (_RB, TT), 0).astype(jnp.float32)
    v, i = acc_v, acc_b * float(_RB) + sub_iota
    for s in (4, 2, 1):
        v1, v2 = v[:s], v[s:]
        i1, i2 = i[:s], i[s:]
        take2 = (v2 < v1) | ((v2 == v1) & (i2 < i1))
        v = jnp.where(take2, v2, v1)
        i = jnp.where(take2, i2, i1)
    out_ref[pl.ds(t, 1), :] = i.astype(jnp.int32)


def _nearest_codes(ft2, cb, fn):
    return pl.pallas_call(
        _argmin_body,
        grid=(NT,),
        in_specs=[
            pl.BlockSpec((TT, D), lambda t: (t, 0)),
            pl.BlockSpec((V, D), lambda t: (0, 0)),
            pl.BlockSpec((NT, TT), lambda t: (0, 0)),
        ],
        out_specs=pl.BlockSpec((NT, TT), lambda t: (0, 0)),
        out_shape=jax.ShapeDtypeStruct((NT, TT), jnp.int32),
        scratch_shapes=[pltpu.VMEM((V, 1), jnp.float32)],
        compiler_params=pltpu.CompilerParams(
            dimension_semantics=("arbitrary",)),
    )(ft2, cb, fn)


_NC = 2
_NS = 16
_NW = _NC * _NS
_ROWS_PER_W = N // _NW
_CHUNK = 72
_NCHUNK = _ROWS_PER_W // _CHUNK


def _gather_body(idx_hbm, cb_hbm, out_hbm, idx_v, rows_v, sem0, sem1):
    wid = lax.axis_index("s") * _NC + lax.axis_index("c")
    base = wid * _ROWS_PER_W
    sems = (sem0, sem1)
    for c in range(_NCHUNK):
        pltpu.sync_copy(idx_hbm.at[pl.ds(base + c * _CHUNK, _CHUNK)],
                        idx_v.at[c])
    prev = pltpu.async_copy(cb_hbm.at[idx_v.at[0]], rows_v.at[0], sems[0])
    for c in range(_NCHUNK):
        nxt = None
        if c + 1 < _NCHUNK:
            nxt = pltpu.async_copy(cb_hbm.at[idx_v.at[c + 1]],
                                   rows_v.at[(c + 1) % 2], sems[(c + 1) % 2])
        prev.wait()
        pltpu.sync_copy(rows_v.at[c % 2],
                        out_hbm.at[pl.ds(base + c * _CHUNK, _CHUNK)])
        prev = nxt


def _gather_rows(idx_flat, cb):
    return pl.kernel(
        _gather_body,
        mesh=plsc.VectorSubcoreMesh(core_axis_name="c", subcore_axis_name="s"),
        out_type=jax.ShapeDtypeStruct((N, D), jnp.float32),
        scratch_types=[
            pltpu.VMEM((_NCHUNK, _CHUNK), jnp.int32),
            pltpu.VMEM((2, _CHUNK, D), jnp.float32),
            pltpu.SemaphoreType.DMA,
            pltpu.SemaphoreType.DMA,
        ],
    )(idx_flat, cb)


def kernel(h, cb):
    ft2 = -2.0 * jnp.transpose(h, (0, 2, 3, 1)).reshape(N, D)
    fn = (jnp.sum(ft2 * ft2, axis=1) * 0.25).reshape(NT, TT)
    idx = _nearest_codes(ft2, cb, fn)
    idx_flat = idx.reshape(N)
    q = _gather_rows(idx_flat, cb)
    z = idx_flat.reshape(B, H, W)
    return (z, q.reshape(B, H, W, D))
